# baseline (speedup 1.0000x reference)
import functools

import jax
import jax.numpy as jnp
from jax import lax
from jax.experimental import pallas as pl
from jax.experimental.pallas import tpu as pltpu
from jax.experimental.pallas import tpu_sc as plsc

_ORDER = 16
_LANES = 16


def _build_sc_gather(shape2d, num_workers: int, col_chunk: int):
    rows, cols = shape2d
    cols_per_w = cols // num_workers
    nchunks = cols_per_w // col_chunk
    slices_per_row = col_chunk // _LANES

    mesh = plsc.VectorSubcoreMesh(core_axis_name="c", subcore_axis_name="s")

    @functools.partial(
        pl.kernel,
        mesh=mesh,
        out_type=jax.ShapeDtypeStruct(shape2d, jnp.float32),
        scratch_types=[
            pltpu.VMEM((_ORDER,), jnp.float32),
            pltpu.VMEM((2, rows, col_chunk), jnp.int32),
            pltpu.VMEM((2, rows, col_chunk), jnp.float32),
            pltpu.SemaphoreType.DMA((2,)),
            pltpu.SemaphoreType.DMA((2,)),
        ],
        compiler_params=pltpu.CompilerParams(
            needs_layout_passes=False, skip_device_barrier=True
        ),
    )
    def sc_gather(idx_hbm, tbl_hbm, out_hbm, tbl_v, idx_v, out_v, isem, osem):
        wid = lax.axis_index("s") * 2 + lax.axis_index("c")
        col0 = wid * cols_per_w
        pltpu.sync_copy(tbl_hbm, tbl_v)
        tbl = tbl_v[...]

        def idx_copy(c, buf):
            return pltpu.make_async_copy(
                idx_hbm.at[:, pl.ds(col0 + c * col_chunk, col_chunk)],
                idx_v.at[buf],
                isem.at[buf],
            )

        def out_copy(c, buf):
            return pltpu.make_async_copy(
                out_v.at[buf],
                out_hbm.at[:, pl.ds(col0 + c * col_chunk, col_chunk)],
                osem.at[buf],
            )

        idx_copy(0, 0).start()
        for c in range(nchunks):
            buf = c % 2
            if c + 1 < nchunks:
                idx_copy(c + 1, 1 - buf).start()
            idx_copy(c, buf).wait()
            if c >= 2:
                out_copy(c - 2, buf).wait()

            @plsc.parallel_loop(0, rows, 1, unroll=4)
            def vec_body(r):
                for j in range(slices_per_row):
                    sl = pl.ds(j * _LANES, _LANES)
                    out_v[buf, r, sl] = jnp.take_along_axis(
                        tbl, idx_v[buf, r, sl], axis=0,
                        mode="promise_in_bounds",
                    )

            out_copy(c, buf).start()
        out_copy(nchunks - 2, nchunks % 2).wait()
        out_copy(nchunks - 1, (nchunks - 1) % 2).wait()

    return sc_gather


def kernel(input, val_table):
    inp_t = input.T
    fn = _build_sc_gather(inp_t.shape, num_workers=32, col_chunk=128)
    out_t = fn(inp_t, val_table)
    return out_t.T


# speedup vs baseline: 1.0076x; 1.0076x over previous
import functools

import jax
import jax.numpy as jnp
from jax import lax
from jax.experimental import pallas as pl
from jax.experimental.pallas import tpu as pltpu
from jax.experimental.pallas import tpu_sc as plsc

_ORDER = 16
_LANES = 16


def _build_sc_gather(shape2d, num_workers: int, col_chunk: int):
    rows, cols = shape2d
    cols_per_w = cols // num_workers
    nchunks = cols_per_w // col_chunk
    slices_per_row = col_chunk // _LANES

    mesh = plsc.VectorSubcoreMesh(core_axis_name="c", subcore_axis_name="s")

    @functools.partial(
        pl.kernel,
        mesh=mesh,
        out_type=jax.ShapeDtypeStruct(shape2d, jnp.float32),
        scratch_types=[
            pltpu.VMEM((_ORDER,), jnp.float32),
            pltpu.VMEM((2, rows, col_chunk), jnp.int32),
            pltpu.VMEM((2, rows, col_chunk), jnp.float32),
            pltpu.SemaphoreType.DMA((2,)),
            pltpu.SemaphoreType.DMA((2,)),
        ],
        compiler_params=pltpu.CompilerParams(
            needs_layout_passes=False, skip_device_barrier=True
        ),
    )
    def sc_gather(idx_hbm, tbl_hbm, out_hbm, tbl_v, idx_v, out_v, isem, osem):
        wid = lax.axis_index("s") * 2 + lax.axis_index("c")
        col0 = wid * cols_per_w
        pltpu.sync_copy(tbl_hbm, tbl_v)
        tbl = tbl_v[...]

        def idx_copy(c, buf):
            return pltpu.make_async_copy(
                idx_hbm.at[:, pl.ds(col0 + c * col_chunk, col_chunk)],
                idx_v.at[buf],
                isem.at[buf],
            )

        def out_copy(c, buf):
            return pltpu.make_async_copy(
                out_v.at[buf],
                out_hbm.at[:, pl.ds(col0 + c * col_chunk, col_chunk)],
                osem.at[buf],
            )

        idx_copy(0, 0).start()
        for c in range(nchunks):
            buf = c % 2
            if c + 1 < nchunks:
                idx_copy(c + 1, 1 - buf).start()
            idx_copy(c, buf).wait()
            if c >= 2:
                out_copy(c - 2, buf).wait()

            @plsc.parallel_loop(0, rows, 1, unroll=2)
            def vec_body(r):
                for j in range(slices_per_row):
                    sl = pl.ds(j * _LANES, _LANES)
                    out_v[buf, r, sl] = jnp.take_along_axis(
                        tbl, idx_v[buf, r, sl], axis=0,
                        mode="promise_in_bounds",
                    )

            out_copy(c, buf).start()
        out_copy(nchunks - 2, nchunks % 2).wait()
        out_copy(nchunks - 1, (nchunks - 1) % 2).wait()

    return sc_gather


def kernel(input, val_table):
    inp_t = input.T
    fn = _build_sc_gather(inp_t.shape, num_workers=32, col_chunk=128)
    out_t = fn(inp_t, val_table)
    return out_t.T


# speedup vs baseline: 1.0177x; 1.0100x over previous
import functools

import jax
import jax.numpy as jnp
from jax import lax
from jax.experimental import pallas as pl
from jax.experimental.pallas import tpu as pltpu
from jax.experimental.pallas import tpu_sc as plsc

_ORDER = 16
_LANES = 16


def _build_sc_gather(shape2d, num_workers: int, col_chunk: int):
    rows, cols = shape2d
    cols_per_w = cols // num_workers
    nchunks = cols_per_w // col_chunk
    slices_per_row = col_chunk // _LANES

    mesh = plsc.VectorSubcoreMesh(core_axis_name="c", subcore_axis_name="s")

    @functools.partial(
        pl.kernel,
        mesh=mesh,
        out_type=jax.ShapeDtypeStruct(shape2d, jnp.float32),
        scratch_types=[
            pltpu.VMEM((_ORDER,), jnp.float32),
            pltpu.VMEM((2, rows, col_chunk), jnp.int32),
            pltpu.VMEM((2, rows, col_chunk), jnp.float32),
            pltpu.SemaphoreType.DMA((2,)),
            pltpu.SemaphoreType.DMA((2,)),
        ],
        compiler_params=pltpu.CompilerParams(
            needs_layout_passes=False, skip_device_barrier=True
        ),
    )
    def sc_gather(idx_hbm, tbl_hbm, out_hbm, tbl_v, idx_v, out_v, isem, osem):
        wid = lax.axis_index("s") * 2 + lax.axis_index("c")
        col0 = wid * cols_per_w
        pltpu.sync_copy(tbl_hbm, tbl_v)
        tbl = tbl_v[...]

        def idx_copy(c, buf):
            return pltpu.make_async_copy(
                idx_hbm.at[:, pl.ds(col0 + c * col_chunk, col_chunk)],
                idx_v.at[buf],
                isem.at[buf],
            )

        def out_copy(c, buf):
            return pltpu.make_async_copy(
                out_v.at[buf],
                out_hbm.at[:, pl.ds(col0 + c * col_chunk, col_chunk)],
                osem.at[buf],
            )

        idx_copy(0, 0).start()

        def chunk_body(c, carry):
            buf = lax.rem(c, 2)

            @pl.when(c + 1 < nchunks)
            def _():
                idx_copy(c + 1, 1 - buf).start()

            idx_copy(c, buf).wait()

            @pl.when(c >= 2)
            def _():
                out_copy(c - 2, buf).wait()

            @plsc.parallel_loop(0, rows, 1, unroll=2)
            def vec_body(r):
                for j in range(slices_per_row):
                    sl = pl.ds(j * _LANES, _LANES)
                    out_v[buf, r, sl] = jnp.take_along_axis(
                        tbl, idx_v[buf, r, sl], axis=0,
                        mode="promise_in_bounds",
                    )

            out_copy(c, buf).start()
            return carry

        lax.fori_loop(0, nchunks, chunk_body, 0)
        out_copy(nchunks - 2, nchunks % 2).wait()
        out_copy(nchunks - 1, (nchunks - 1) % 2).wait()

    return sc_gather


def kernel(input, val_table):
    inp_t = input.T
    fn = _build_sc_gather(inp_t.shape, num_workers=32, col_chunk=128)
    out_t = fn(inp_t, val_table)
    return out_t.T


# speedup vs baseline: 1.0685x; 1.0499x over previous
import functools

import jax
import jax.numpy as jnp
from jax import lax
from jax.experimental import pallas as pl
from jax.experimental.pallas import tpu as pltpu
from jax.experimental.pallas import tpu_sc as plsc

_ORDER = 16
_LANES = 16


def _build_sc_gather(shape2d, num_workers: int, col_chunk: int):
    rows, cols = shape2d
    cols_per_w = cols // num_workers
    nchunks = cols_per_w // col_chunk
    assert nchunks == 2, "pipeline below primes both buffers up front"
    slices_per_row = col_chunk // _LANES

    mesh = plsc.VectorSubcoreMesh(core_axis_name="c", subcore_axis_name="s")

    @functools.partial(
        pl.kernel,
        mesh=mesh,
        out_type=jax.ShapeDtypeStruct(shape2d, jnp.float32),
        scratch_types=[
            pltpu.VMEM((_ORDER,), jnp.float32),
            pltpu.VMEM((2, rows, col_chunk), jnp.int32),
            pltpu.SemaphoreType.DMA,
            pltpu.SemaphoreType.DMA((2,)),
            pltpu.SemaphoreType.DMA((2,)),
        ],
        compiler_params=pltpu.CompilerParams(
            needs_layout_passes=False, skip_device_barrier=True
        ),
    )
    def sc_gather(idx_hbm, tbl_hbm, out_hbm, tbl_v, buf_v, tsem, isem, osem):
        wid = lax.axis_index("s") * 2 + lax.axis_index("c")
        col0 = wid * cols_per_w
        val_v = buf_v.bitcast(jnp.float32)

        def idx_copy(c, buf):
            return pltpu.make_async_copy(
                idx_hbm.at[:, pl.ds(col0 + c * col_chunk, col_chunk)],
                buf_v.at[buf],
                isem.at[buf],
            )

        def out_copy(c, buf):
            return pltpu.make_async_copy(
                val_v.at[buf],
                out_hbm.at[:, pl.ds(col0 + c * col_chunk, col_chunk)],
                osem.at[buf],
            )

        tbl_copy = pltpu.make_async_copy(tbl_hbm, tbl_v, tsem)
        tbl_copy.start()
        idx_copy(0, 0).start()
        idx_copy(1, 1).start()
        tbl_copy.wait()
        tbl = tbl_v[...]

        for c in range(nchunks):
            buf = c % 2
            idx_copy(c, buf).wait()
            if c >= 2:
                out_copy(c - 2, buf).wait()

            @plsc.parallel_loop(0, rows, 1, unroll=2)
            def vec_body(r):
                for j in range(slices_per_row):
                    sl = pl.ds(j * _LANES, _LANES)
                    val_v[buf, r, sl] = jnp.take_along_axis(
                        tbl, buf_v[buf, r, sl], axis=0,
                        mode="promise_in_bounds",
                    )

            out_copy(c, buf).start()
        out_copy(nchunks - 2, nchunks % 2).wait()
        out_copy(nchunks - 1, (nchunks - 1) % 2).wait()

    return sc_gather


def kernel(input, val_table):
    inp_t = input.T
    fn = _build_sc_gather(inp_t.shape, num_workers=32, col_chunk=256)
    out_t = fn(inp_t, val_table)
    return out_t.T


# speedup vs baseline: 1.1030x; 1.0323x over previous
import functools

import jax
import jax.numpy as jnp
from jax import lax
from jax.experimental import pallas as pl
from jax.experimental.pallas import tpu as pltpu
from jax.experimental.pallas import tpu_sc as plsc

_ORDER = 16
_LANES = 16


def _build_sc_gather(shape2d, num_workers: int, col_chunk: int):
    rows, cols = shape2d
    cols_per_w = cols // num_workers
    nchunks = cols_per_w // col_chunk
    slices_per_row = col_chunk // _LANES

    mesh = plsc.VectorSubcoreMesh(core_axis_name="c", subcore_axis_name="s")

    @functools.partial(
        pl.kernel,
        mesh=mesh,
        out_type=jax.ShapeDtypeStruct(shape2d, jnp.float32),
        scratch_types=[
            pltpu.VMEM((_ORDER,), jnp.float32),
            pltpu.VMEM((4, rows, col_chunk), jnp.int32),
            pltpu.SemaphoreType.DMA,
            pltpu.SemaphoreType.DMA((4,)),
            pltpu.SemaphoreType.DMA((4,)),
        ],
        compiler_params=pltpu.CompilerParams(
            needs_layout_passes=False, skip_device_barrier=True
        ),
    )
    def sc_gather(idx_hbm, tbl_hbm, out_hbm, tbl_v, buf_v, tsem, isem, osem):
        wid = lax.axis_index("s") * 2 + lax.axis_index("c")
        col0 = wid * cols_per_w
        val_v = buf_v.bitcast(jnp.float32)

        def idx_copy(c, buf):
            return pltpu.make_async_copy(
                idx_hbm.at[:, pl.ds(col0 + c * col_chunk, col_chunk)],
                buf_v.at[buf],
                isem.at[buf],
            )

        def out_copy(c, buf):
            return pltpu.make_async_copy(
                val_v.at[buf],
                out_hbm.at[:, pl.ds(col0 + c * col_chunk, col_chunk)],
                osem.at[buf],
            )

        tbl_copy = pltpu.make_async_copy(tbl_hbm, tbl_v, tsem)
        tbl_copy.start()
        for c in range(nchunks):
            idx_copy(c, c).start()
        tbl_copy.wait()
        tbl = tbl_v[...]

        for c in range(nchunks):
            idx_copy(c, c).wait()

            @plsc.parallel_loop(0, rows, 1, unroll=2)
            def vec_body(r):
                for j in range(slices_per_row):
                    sl = pl.ds(j * _LANES, _LANES)
                    val_v[c, r, sl] = jnp.take_along_axis(
                        tbl, buf_v[c, r, sl], axis=0,
                        mode="promise_in_bounds",
                    )

            out_copy(c, c).start()
        for c in range(nchunks):
            out_copy(c, c).wait()

    return sc_gather


def kernel(input, val_table):
    inp_t = input.T
    fn = _build_sc_gather(inp_t.shape, num_workers=32, col_chunk=128)
    out_t = fn(inp_t, val_table)
    return out_t.T
